# baseline (speedup 1.0000x reference)
import functools

import jax
import jax.numpy as jnp
from jax import lax
from jax.experimental import pallas as pl
from jax.experimental.pallas import tpu as pltpu
from jax.experimental.pallas import tpu_sc as plsc

N_A = 259200
N_B = 131072
NNZ = 524288
BATCH = 32
DST = (256, 512)

NC, NS, L = 2, 16, 16
NW = NC * NS
ROWS_W = N_B // NW
R = 256
CH = ROWS_W // R
G = 4 * R
IW = 128
NSUB = G // IW


def _sc_regrid(x_t, col2d, w):
    mesh = plsc.VectorSubcoreMesh(core_axis_name="c", subcore_axis_name="s")

    @functools.partial(
        pl.kernel,
        out_type=jax.ShapeDtypeStruct((N_B, BATCH), jnp.bfloat16),
        mesh=mesh,
        compiler_params=pltpu.CompilerParams(
            needs_layout_passes=False, use_tc_tiling_on_sc=False),
        scratch_types=[
            pltpu.VMEM((2 * NSUB, IW), jnp.int32),
            pltpu.VMEM((2 * G,), jnp.float32),
            pltpu.VMEM((2 * G, BATCH), jnp.bfloat16),
            pltpu.VMEM((2 * R, BATCH), jnp.bfloat16),
            pltpu.SemaphoreType.DMA,
            pltpu.SemaphoreType.DMA,
            pltpu.SemaphoreType.DMA,
        ],
    )
    def k(x_hbm, col_hbm, w_hbm, out_hbm, col_v, w_v, rows_v, out_v,
          csem, gsem, osem):
        wid = lax.axis_index("s") * NC + lax.axis_index("c")

        def fire_colw(ch, buf):
            base = pl.multiple_of(wid * ROWS_W + ch * R, R)
            crow0 = pl.multiple_of(base // (IW // 4), NSUB)
            pltpu.async_copy(col_hbm.at[pl.ds(crow0, NSUB), :],
                             col_v.at[pl.ds(buf * NSUB, NSUB), :], csem)
            pltpu.async_copy(w_hbm.at[pl.ds(pl.multiple_of(4 * base, G), G)],
                             w_v.at[pl.ds(buf * G, G)], csem)

        def drain_colw():
            pltpu.make_async_copy(col_hbm.at[pl.ds(0, NSUB), :],
                                  col_v.at[pl.ds(0, NSUB), :], csem).wait()
            pltpu.make_async_copy(w_hbm.at[pl.ds(0, G)],
                                  w_v.at[pl.ds(0, G)], csem).wait()

        def fire_gathers(buf):
            for i in range(NSUB):
                pltpu.async_copy(
                    x_hbm.at[col_v.at[buf * NSUB + i]],
                    rows_v.at[pl.ds(buf * G + i * IW, IW)], gsem)

        def drain_gathers():
            for i in range(NSUB):
                pltpu.make_async_copy(x_hbm.at[pl.ds(0, IW)],
                                      rows_v.at[pl.ds(i * IW, IW)],
                                      gsem).wait()

        def drain_out():
            pltpu.make_async_copy(out_v.at[pl.ds(0, R)],
                                  out_hbm.at[pl.ds(0, R)], osem).wait()

        fire_colw(0, 0)
        drain_colw()
        fire_gathers(0)
        fire_colw(1, 1)

        def chunk_body(c, carry):
            cur = lax.bitwise_and(c, 1)
            nxt = 1 - cur
            base = pl.multiple_of(wid * ROWS_W + c * R, R)

            drain_gathers()
            drain_colw()
            fire_gathers(nxt)

            @pl.when(c >= 2)
            def _():
                drain_out()

            UNROLL = 4

            def group_body(j, carry2):
                r0 = cur * R + j * UNROLL
                wv = w_v[pl.ds(4 * r0, 4 * UNROLL)]
                for u in range(UNROLL):
                    r = r0 + u
                    acc0 = None
                    acc1 = None
                    for kk in range(4):
                        wsc = wv[4 * u + kk]
                        va, vb = plsc.unpack(
                            rows_v[4 * r + kk, :],
                            format=plsc.PackFormat.INTERLEAVED)
                        acc0 = va * wsc if acc0 is None else acc0 + va * wsc
                        acc1 = vb * wsc if acc1 is None else acc1 + vb * wsc
                    out_v[r, :] = plsc.pack(
                        acc0, acc1, format=plsc.PackFormat.INTERLEAVED)
                return carry2

            lax.fori_loop(0, R // UNROLL, group_body, 0)

            pltpu.async_copy(out_v.at[pl.ds(cur * R, R)],
                             out_hbm.at[pl.ds(base, R)], osem)
            nxt2 = lax.rem(c + 2, CH)
            fire_colw(nxt2, cur)
            return carry

        lax.fori_loop(0, CH, chunk_body, 0)

        drain_gathers()
        drain_colw()
        drain_out()
        drain_out()

    return k(x_t, col2d, w)


def kernel(x, row, col, weights):
    del row
    x_t = x.astype(jnp.bfloat16).reshape(BATCH, N_A).T
    col2 = col + (258480 - 1440 * (col // 720))
    y = _sc_regrid(x_t, col2.reshape(-1, IW), weights)
    return y.T.astype(jnp.float32).reshape(BATCH, *DST)


# speedup vs baseline: 1.9065x; 1.9065x over previous
import functools

import jax
import jax.numpy as jnp
from jax import lax
from jax.experimental import pallas as pl
from jax.experimental.pallas import tpu as pltpu
from jax.experimental.pallas import tpu_sc as plsc

N_A = 259200
N_B = 131072
NNZ = 524288
BATCH = 32
DST = (256, 512)

NC, NS, L = 2, 16, 16
NW = NC * NS
ROWS_W = N_B // NW
R = 256
CH = ROWS_W // R
G = 4 * R
IW = 128
NSUB = G // IW
RP = R + 1


def _sc_regrid(x_t, col2d, w):
    mesh = plsc.VectorSubcoreMesh(core_axis_name="c", subcore_axis_name="s")

    @functools.partial(
        pl.kernel,
        out_type=jax.ShapeDtypeStruct((BATCH, N_B), jnp.float32),
        mesh=mesh,
        compiler_params=pltpu.CompilerParams(
            needs_layout_passes=False, use_tc_tiling_on_sc=False),
        scratch_types=[
            pltpu.VMEM((2 * NSUB, IW), jnp.int32),
            pltpu.VMEM((2 * G,), jnp.float32),
            pltpu.VMEM((2 * G, BATCH), jnp.float32),
            pltpu.VMEM((2 * BATCH, RP), jnp.float32),
            pltpu.SemaphoreType.DMA,
            pltpu.SemaphoreType.DMA,
            pltpu.SemaphoreType.DMA,
        ],
    )
    def k(x_hbm, col_hbm, w_hbm, out_hbm, col_v, w_v, rows_v, out_v,
          csem, gsem, osem):
        wid = lax.axis_index("s") * NC + lax.axis_index("c")

        def fire_colw(ch, buf):
            base = pl.multiple_of(wid * ROWS_W + ch * R, R)
            crow0 = pl.multiple_of(base // (IW // 4), NSUB)
            pltpu.async_copy(col_hbm.at[pl.ds(crow0, NSUB), :],
                             col_v.at[pl.ds(buf * NSUB, NSUB), :], csem)
            pltpu.async_copy(w_hbm.at[pl.ds(pl.multiple_of(4 * base, G), G)],
                             w_v.at[pl.ds(buf * G, G)], csem)

        def drain_colw():
            pltpu.make_async_copy(col_hbm.at[pl.ds(0, NSUB), :],
                                  col_v.at[pl.ds(0, NSUB), :], csem).wait()
            pltpu.make_async_copy(w_hbm.at[pl.ds(0, G)],
                                  w_v.at[pl.ds(0, G)], csem).wait()

        def fire_gathers(buf):
            for i in range(NSUB):
                pltpu.async_copy(
                    x_hbm.at[col_v.at[buf * NSUB + i]],
                    rows_v.at[pl.ds(buf * G + i * IW, IW)], gsem)

        def drain_gathers():
            for i in range(NSUB):
                pltpu.make_async_copy(x_hbm.at[pl.ds(0, IW)],
                                      rows_v.at[pl.ds(i * IW, IW)],
                                      gsem).wait()

        def drain_out():
            pltpu.make_async_copy(out_v.at[pl.ds(0, BATCH), pl.ds(0, R)],
                                  out_hbm.at[:, pl.ds(0, R)], osem).wait()

        fire_colw(0, 0)
        drain_colw()
        fire_gathers(0)
        fire_colw(1, 1)

        def chunk_body(c, carry):
            cur = lax.bitwise_and(c, 1)
            nxt = 1 - cur
            base = pl.multiple_of(wid * ROWS_W + c * R, R)

            drain_gathers()
            drain_colw()
            fire_gathers(nxt)

            @pl.when(c >= 2)
            def _():
                drain_out()

            UNROLL = 4
            brow0 = cur * BATCH + lax.iota(jnp.int32, L)
            brow1 = brow0 + L

            def group_body(j, carry2):
                g0 = cur * G + 4 * UNROLL * j
                wv = w_v[pl.ds(g0, 4 * UNROLL)]
                rcol = j * UNROLL
                for u in range(UNROLL):
                    acc0 = None
                    acc1 = None
                    for kk in range(4):
                        wsc = wv[4 * u + kk]
                        i = g0 + 4 * u + kk
                        src = rows_v[i, pl.ds(0, L)] * wsc
                        src1 = rows_v[i, pl.ds(L, L)] * wsc
                        acc0 = src if acc0 is None else acc0 + src
                        acc1 = src1 if acc1 is None else acc1 + src1
                    cvec = jnp.full((L,), rcol + u, jnp.int32)
                    plsc.store_scatter(out_v, [brow0, cvec], acc0)
                    plsc.store_scatter(out_v, [brow1, cvec], acc1)
                return carry2

            lax.fori_loop(0, R // UNROLL, group_body, 0)

            pltpu.async_copy(out_v.at[pl.ds(cur * BATCH, BATCH), pl.ds(0, R)],
                             out_hbm.at[:, pl.ds(base, R)], osem)
            nxt2 = lax.rem(c + 2, CH)
            fire_colw(nxt2, cur)
            return carry

        lax.fori_loop(0, CH, chunk_body, 0)

        drain_gathers()
        drain_colw()
        drain_out()
        drain_out()

    return k(x_t, col2d, w)


def kernel(x, row, col, weights):
    del row
    x_t = x.reshape(BATCH, N_A).T
    col2 = col + (258480 - 1440 * (col // 720))
    y = _sc_regrid(x_t, col2.reshape(-1, IW), weights)
    return y.reshape(BATCH, *DST)
